# Initial kernel scaffold; baseline (speedup 1.0000x reference)
import jax
import jax.numpy as jnp
from jax.experimental import pallas as pl


def kernel(x, edge_index, W1, b1, W2, b2):
    raise NotImplementedError("write your pallas kernel here")


# speedup vs baseline: 14.2599x; 14.2599x over previous
import functools

import jax
import jax.numpy as jnp
from jax import lax
from jax.experimental import pallas as pl
from jax.experimental.pallas import tpu as pltpu
from jax.experimental.pallas import tpu_sc as plsc

NC = 2
NS = 16
NW = NC * NS
B = 100
G = 10
HW = 128

@functools.cache
def _sc_mesh():
  return plsc.VectorSubcoreMesh(
      core_axis_name="c", subcore_axis_name="s", num_cores=NC, num_subcores=NS
  )


def _zero_rows(zeros_v, acc_sh, base, rps):
  full, rem = divmod(rps, B)
  for k in range(full):
    pltpu.sync_copy(zeros_v, acc_sh.at[pl.ds(base + k * B, B)])
  if rem:
    pltpu.sync_copy(zeros_v.at[pl.ds(0, rem)],
                    acc_sh.at[pl.ds(base + full * B, rem)])


def _make_hist_kernel(n_nodes, ng):
  rps = n_nodes // NS

  @functools.partial(
      pl.kernel,
      out_type=jax.ShapeDtypeStruct((NC, NS, rps, HW), jnp.float32),
      mesh=_sc_mesh(),
      scratch_types=[
          pltpu.VMEM((G, B), jnp.int32),
          pltpu.VMEM((B, HW), jnp.float32),
          pltpu.VMEM_SHARED((n_nodes, HW), jnp.float32),
      ],
  )
  def hist_kernel(dst_hbm, ones_hbm, zeros_hbm, out_hbm, idx_v, rows_v,
                  hist_sh):
    cid = lax.axis_index("c")
    sid = lax.axis_index("s")
    wid = sid * NC + cid
    pltpu.sync_copy(zeros_hbm, rows_v)
    _zero_rows(rows_v, hist_sh, sid * rps, rps)
    pltpu.sync_copy(ones_hbm, rows_v)
    plsc.subcore_barrier()

    def body(gi, carry):
      pltpu.sync_copy(dst_hbm.at[wid, gi], idx_v)

      def inner(jj, carry):
        pltpu.sync_copy(rows_v, hist_sh.at[idx_v.at[jj]], add=True)
        return carry

      return lax.fori_loop(0, G, inner, carry)

    lax.fori_loop(0, ng, body, 0)
    plsc.subcore_barrier()
    pltpu.sync_copy(hist_sh.at[pl.ds(sid * rps, rps)], out_hbm.at[cid, sid])

  return hist_kernel


def _make_agg_kernel(n_nodes, ng, nch, d):
  rps = n_nodes // NS

  @functools.partial(
      pl.kernel,
      out_type=[jax.ShapeDtypeStruct((NC, NS, rps, d), jnp.float32)] * nch,
      mesh=_sc_mesh(),
      scratch_types=[
          pltpu.VMEM((G, B), jnp.int32),
          pltpu.VMEM((G, B), jnp.int32),
          pltpu.VMEM((B, d), jnp.float32),
          pltpu.VMEM_SHARED((n_nodes, d), jnp.float32),
          pltpu.SemaphoreType.DMA,
      ],
  )
  def agg_kernel(*refs):
    g_refs = refs[:nch]
    src_hbm, dst_hbm, zeros_hbm = refs[nch:nch + 3]
    out_refs = refs[nch + 3:2 * nch + 3]
    idxs_v, idxd_v, rows_v, acc_sh, sem = refs[2 * nch + 3:]
    cid = lax.axis_index("c")
    sid = lax.axis_index("s")
    wid = sid * NC + cid
    for c in range(nch):
      pltpu.sync_copy(zeros_hbm, rows_v)
      _zero_rows(rows_v, acc_sh, sid * rps, rps)
      plsc.subcore_barrier()

      def body(gi, carry):
        pltpu.sync_copy(src_hbm.at[wid, gi], idxs_v)
        pltpu.sync_copy(dst_hbm.at[wid, gi], idxd_v)

        def inner(jj, carry):
          pltpu.async_copy(g_refs[c].at[idxs_v.at[jj]], rows_v, sem).wait()
          pltpu.sync_copy(rows_v, acc_sh.at[idxd_v.at[jj]], add=True)
          return carry

        return lax.fori_loop(0, G, inner, carry)

      lax.fori_loop(0, ng, body, 0)
      plsc.subcore_barrier()
      pltpu.sync_copy(acc_sh.at[pl.ds(sid * rps, rps)],
                      out_refs[c].at[cid, sid])
      plsc.subcore_barrier()

  return agg_kernel


def _dinv_block(hist_ref):
  deg = hist_ref[0] + hist_ref[1]
  return lax.rsqrt(deg[:, 0:1] + 1.0)


def _g1_body(hist_ref, x_ref, w1_ref, ga_ref, gb_ref):
  dinv = _dinv_block(hist_ref)
  h = jnp.dot(x_ref[...], w1_ref[...], preferred_element_type=jnp.float32)
  ga_ref[...] = h[:, :128] * dinv
  gb_ref[...] = h[:, 128:] * dinv


def _g2_body(hist_ref, a0_ref, a1_ref, ga_ref, gb_ref, w2_ref, b1_ref,
             g2_ref):
  dinv = _dinv_block(hist_ref)
  h0 = jnp.maximum(dinv * (a0_ref[0] + a0_ref[1] + ga_ref[...])
                   + b1_ref[0:1, :128], 0.0)
  h1 = jnp.maximum(dinv * (a1_ref[0] + a1_ref[1] + gb_ref[...])
                   + b1_ref[0:1, 128:], 0.0)
  g2_ref[...] = dinv * (
      jnp.dot(h0, w2_ref[0], preferred_element_type=jnp.float32)
      + jnp.dot(h1, w2_ref[1], preferred_element_type=jnp.float32))


def _out_body(hist_ref, a2_ref, g2_ref, b2_ref, out_ref):
  dinv = _dinv_block(hist_ref)
  out_ref[...] = dinv * (a2_ref[0] + a2_ref[1] + g2_ref[...]) + b2_ref[...]


def kernel(x, edge_index, W1, b1, W2, b2):
  n, d_in = x.shape
  d_hid = W1.shape[1]
  d_out = W2.shape[1]
  e = edge_index.shape[1]
  assert e % (NW * B * G) == 0 and n % NS == 0
  assert d_in == 128 and d_hid == 256 and d_out == 128
  ng = e // (NW * B * G)
  rb = 1000
  nblk = n // rb

  src = edge_index[0].reshape(NW, ng, G, B)
  dst = edge_index[1].reshape(NW, ng, G, B)
  ones_h = jnp.ones((B, HW), jnp.float32)
  zeros_d = jnp.zeros((B, 128), jnp.float32)
  w2r = W2.reshape(2, 128, d_out)
  b1r = b1.reshape(1, d_hid)
  b2r = b2.reshape(1, d_out)

  hist = _make_hist_kernel(n, ng)(dst, ones_h, zeros_d).reshape(NC, n, HW)

  hist_spec = pl.BlockSpec((NC, rb, HW), lambda i: (0, i, 0))
  row_spec = pl.BlockSpec((rb, 128), lambda i: (i, 0))
  agg_spec = pl.BlockSpec((NC, rb, 128), lambda i: (0, i, 0))

  g1a, g1b = pl.pallas_call(
      _g1_body,
      grid=(nblk,),
      in_specs=[
          hist_spec,
          pl.BlockSpec((rb, d_in), lambda i: (i, 0)),
          pl.BlockSpec((d_in, d_hid), lambda i: (0, 0)),
      ],
      out_specs=[row_spec, row_spec],
      out_shape=[jax.ShapeDtypeStruct((n, 128), jnp.float32)] * 2,
  )(hist, x, W1)

  a1a, a1b = _make_agg_kernel(n, ng, 2, 128)(g1a, g1b, src, dst, zeros_d)
  a1a = a1a.reshape(NC, n, 128)
  a1b = a1b.reshape(NC, n, 128)

  g2 = pl.pallas_call(
      _g2_body,
      grid=(nblk,),
      in_specs=[
          hist_spec, agg_spec, agg_spec, row_spec, row_spec,
          pl.BlockSpec((2, 128, d_out), lambda i: (0, 0, 0)),
          pl.BlockSpec((1, d_hid), lambda i: (0, 0)),
      ],
      out_specs=row_spec,
      out_shape=jax.ShapeDtypeStruct((n, 128), jnp.float32),
  )(hist, a1a, a1b, g1a, g1b, w2r, b1r)

  (a2,) = _make_agg_kernel(n, ng, 1, 128)(g2, src, dst, zeros_d)
  a2 = a2.reshape(NC, n, 128)

  out = pl.pallas_call(
      _out_body,
      grid=(nblk,),
      in_specs=[
          hist_spec, agg_spec, row_spec,
          pl.BlockSpec((1, d_out), lambda i: (0, 0)),
      ],
      out_specs=pl.BlockSpec((rb, d_out), lambda i: (i, 0)),
      out_shape=jax.ShapeDtypeStruct((n, d_out), jnp.float32),
  )(hist, a2, g2, b2r)
  return out


# speedup vs baseline: 17.1625x; 1.2035x over previous
import functools

import jax
import jax.numpy as jnp
from jax import lax
from jax.experimental import pallas as pl
from jax.experimental.pallas import tpu as pltpu
from jax.experimental.pallas import tpu_sc as plsc

NC = 2
NS = 16
NW = NC * NS
B = 100
G = 10
HW = 128

@functools.cache
def _sc_mesh():
  return plsc.VectorSubcoreMesh(
      core_axis_name="c", subcore_axis_name="s", num_cores=NC, num_subcores=NS
  )


def _zero_rows(zeros_v, acc_sh, base, rps):
  full, rem = divmod(rps, B)
  for k in range(full):
    pltpu.sync_copy(zeros_v, acc_sh.at[pl.ds(base + k * B, B)])
  if rem:
    pltpu.sync_copy(zeros_v.at[pl.ds(0, rem)],
                    acc_sh.at[pl.ds(base + full * B, rem)])


def _make_hist_kernel(n_nodes, ng):
  rps = n_nodes // NS

  @functools.partial(
      pl.kernel,
      out_type=jax.ShapeDtypeStruct((NC, NS, rps, HW), jnp.float32),
      mesh=_sc_mesh(),
      scratch_types=[
          pltpu.VMEM((G, B), jnp.int32),
          pltpu.VMEM((B, HW), jnp.float32),
          pltpu.VMEM_SHARED((n_nodes, HW), jnp.float32),
          pltpu.SemaphoreType.DMA,
      ],
  )
  def hist_kernel(dst_hbm, ones_hbm, zeros_hbm, out_hbm, idx_v, rows_v,
                  hist_sh, sem):
    cid = lax.axis_index("c")
    sid = lax.axis_index("s")
    wid = sid * NC + cid
    pltpu.sync_copy(zeros_hbm, rows_v)
    _zero_rows(rows_v, hist_sh, sid * rps, rps)
    pltpu.sync_copy(ones_hbm, rows_v)
    plsc.subcore_barrier()

    def body(gi, carry):
      pltpu.sync_copy(dst_hbm.at[wid, gi], idx_v)
      descs = [
          pltpu.async_copy(rows_v, hist_sh.at[idx_v.at[k]], sem, add=True)
          for k in range(G)
      ]
      for dsc in descs:
        dsc.wait()
      return carry

    lax.fori_loop(0, ng, body, 0)
    plsc.subcore_barrier()
    pltpu.sync_copy(hist_sh.at[pl.ds(sid * rps, rps)], out_hbm.at[cid, sid])

  return hist_kernel


def _make_agg_kernel(n_nodes, ng, nch, d):
  rps = n_nodes // NS

  @functools.partial(
      pl.kernel,
      out_type=[jax.ShapeDtypeStruct((NC, NS, rps, d), jnp.float32)] * nch,
      mesh=_sc_mesh(),
      scratch_types=[
          pltpu.VMEM((G, B), jnp.int32),
          pltpu.VMEM((G, B), jnp.int32),
          pltpu.VMEM((B, d), jnp.float32),
          pltpu.VMEM((B, d), jnp.float32),
          pltpu.VMEM_SHARED((n_nodes, d), jnp.float32),
          pltpu.SemaphoreType.DMA,
          pltpu.SemaphoreType.DMA,
          pltpu.SemaphoreType.DMA,
      ],
  )
  def agg_kernel(*refs):
    g_refs = refs[:nch]
    src_hbm, dst_hbm, zeros_hbm = refs[nch:nch + 3]
    out_refs = refs[nch + 3:2 * nch + 3]
    idxs_v, idxd_v, rows0, rows1, acc_sh, gsem, ssem0, ssem1 = (
        refs[2 * nch + 3:])
    rows = [rows0, rows1]
    ssems = [ssem0, ssem1]
    cid = lax.axis_index("c")
    sid = lax.axis_index("s")
    wid = sid * NC + cid
    for c in range(nch):
      pltpu.sync_copy(zeros_hbm, rows0)
      _zero_rows(rows0, acc_sh, sid * rps, rps)
      plsc.subcore_barrier()

      def body(gi, carry):
        pltpu.sync_copy(src_hbm.at[wid, gi], idxs_v)
        pltpu.sync_copy(dst_hbm.at[wid, gi], idxd_v)
        gd = [None] * G
        sd = [None, None]
        gd[0] = pltpu.async_copy(g_refs[c].at[idxs_v.at[0]], rows[0], gsem)
        for k in range(G):
          p = k % 2
          gd[k].wait()
          sd[p] = pltpu.async_copy(rows[p], acc_sh.at[idxd_v.at[k]],
                                   ssems[p], add=True)
          if k + 1 < G:
            q = (k + 1) % 2
            if sd[q] is not None:
              sd[q].wait()
            gd[k + 1] = pltpu.async_copy(g_refs[c].at[idxs_v.at[k + 1]],
                                         rows[q], gsem)
        sd[0].wait()
        sd[1].wait()
        return carry

      lax.fori_loop(0, ng, body, 0)
      plsc.subcore_barrier()
      pltpu.sync_copy(acc_sh.at[pl.ds(sid * rps, rps)],
                      out_refs[c].at[cid, sid])
      plsc.subcore_barrier()

  return agg_kernel


def _dinv_block(hist_ref):
  deg = hist_ref[0] + hist_ref[1]
  return lax.rsqrt(deg[:, 0:1] + 1.0)


def _g1_body(hist_ref, x_ref, w1_ref, ga_ref, gb_ref):
  dinv = _dinv_block(hist_ref)
  h = jnp.dot(x_ref[...], w1_ref[...], preferred_element_type=jnp.float32)
  ga_ref[...] = h[:, :128] * dinv
  gb_ref[...] = h[:, 128:] * dinv


def _g2_body(hist_ref, a0_ref, a1_ref, ga_ref, gb_ref, w2_ref, b1_ref,
             g2_ref):
  dinv = _dinv_block(hist_ref)
  h0 = jnp.maximum(dinv * (a0_ref[0] + a0_ref[1] + ga_ref[...])
                   + b1_ref[0:1, :128], 0.0)
  h1 = jnp.maximum(dinv * (a1_ref[0] + a1_ref[1] + gb_ref[...])
                   + b1_ref[0:1, 128:], 0.0)
  g2_ref[...] = dinv * (
      jnp.dot(h0, w2_ref[0], preferred_element_type=jnp.float32)
      + jnp.dot(h1, w2_ref[1], preferred_element_type=jnp.float32))


def _out_body(hist_ref, a2_ref, g2_ref, b2_ref, out_ref):
  dinv = _dinv_block(hist_ref)
  out_ref[...] = dinv * (a2_ref[0] + a2_ref[1] + g2_ref[...]) + b2_ref[...]


def kernel(x, edge_index, W1, b1, W2, b2):
  n, d_in = x.shape
  d_hid = W1.shape[1]
  d_out = W2.shape[1]
  e = edge_index.shape[1]
  assert e % (NW * B * G) == 0 and n % NS == 0
  assert d_in == 128 and d_hid == 256 and d_out == 128
  ng = e // (NW * B * G)
  rb = 1000
  nblk = n // rb

  src = edge_index[0].reshape(NW, ng, G, B)
  dst = edge_index[1].reshape(NW, ng, G, B)
  ones_h = jnp.ones((B, HW), jnp.float32)
  zeros_d = jnp.zeros((B, 128), jnp.float32)
  w2r = W2.reshape(2, 128, d_out)
  b1r = b1.reshape(1, d_hid)
  b2r = b2.reshape(1, d_out)

  hist = _make_hist_kernel(n, ng)(dst, ones_h, zeros_d).reshape(NC, n, HW)

  hist_spec = pl.BlockSpec((NC, rb, HW), lambda i: (0, i, 0))
  row_spec = pl.BlockSpec((rb, 128), lambda i: (i, 0))
  agg_spec = pl.BlockSpec((NC, rb, 128), lambda i: (0, i, 0))

  g1a, g1b = pl.pallas_call(
      _g1_body,
      grid=(nblk,),
      in_specs=[
          hist_spec,
          pl.BlockSpec((rb, d_in), lambda i: (i, 0)),
          pl.BlockSpec((d_in, d_hid), lambda i: (0, 0)),
      ],
      out_specs=[row_spec, row_spec],
      out_shape=[jax.ShapeDtypeStruct((n, 128), jnp.float32)] * 2,
  )(hist, x, W1)

  a1a, a1b = _make_agg_kernel(n, ng, 2, 128)(g1a, g1b, src, dst, zeros_d)
  a1a = a1a.reshape(NC, n, 128)
  a1b = a1b.reshape(NC, n, 128)

  g2 = pl.pallas_call(
      _g2_body,
      grid=(nblk,),
      in_specs=[
          hist_spec, agg_spec, agg_spec, row_spec, row_spec,
          pl.BlockSpec((2, 128, d_out), lambda i: (0, 0, 0)),
          pl.BlockSpec((1, d_hid), lambda i: (0, 0)),
      ],
      out_specs=row_spec,
      out_shape=jax.ShapeDtypeStruct((n, 128), jnp.float32),
  )(hist, a1a, a1b, g1a, g1b, w2r, b1r)

  (a2,) = _make_agg_kernel(n, ng, 1, 128)(g2, src, dst, zeros_d)
  a2 = a2.reshape(NC, n, 128)

  out = pl.pallas_call(
      _out_body,
      grid=(nblk,),
      in_specs=[
          hist_spec, agg_spec, row_spec,
          pl.BlockSpec((1, d_out), lambda i: (0, 0)),
      ],
      out_specs=pl.BlockSpec((rb, d_out), lambda i: (i, 0)),
      out_shape=jax.ShapeDtypeStruct((n, d_out), jnp.float32),
  )(hist, a2, g2, b2r)
  return out


# speedup vs baseline: 18.1468x; 1.0574x over previous
import functools

import jax
import jax.numpy as jnp
from jax import lax
from jax.experimental import pallas as pl
from jax.experimental.pallas import tpu as pltpu
from jax.experimental.pallas import tpu_sc as plsc

NC = 2
NS = 16
NW = NC * NS
B = 100
G = 10
HW = 128

@functools.cache
def _sc_mesh():
  return plsc.VectorSubcoreMesh(
      core_axis_name="c", subcore_axis_name="s", num_cores=NC, num_subcores=NS
  )


def _zero_rows(zeros_v, acc_sh, base, rps):
  full, rem = divmod(rps, B)
  for k in range(full):
    pltpu.sync_copy(zeros_v, acc_sh.at[pl.ds(base + k * B, B)])
  if rem:
    pltpu.sync_copy(zeros_v.at[pl.ds(0, rem)],
                    acc_sh.at[pl.ds(base + full * B, rem)])


def _make_hist_kernel(n_nodes, ng):
  rps = n_nodes // NS

  @functools.partial(
      pl.kernel,
      out_type=jax.ShapeDtypeStruct((NC, NS, rps, HW), jnp.float32),
      mesh=_sc_mesh(),
      scratch_types=[
          pltpu.VMEM((G, B), jnp.int32),
          pltpu.VMEM((G, B), jnp.int32),
          pltpu.VMEM((B, HW), jnp.float32),
          pltpu.VMEM_SHARED((n_nodes, HW), jnp.float32),
          pltpu.SemaphoreType.DMA,
          pltpu.SemaphoreType.DMA,
      ],
  )
  def hist_kernel(dst_hbm, ones_hbm, zeros_hbm, out_hbm, idx0, idx1, rows_v,
                  hist_sh, sem, isem):
    cid = lax.axis_index("c")
    sid = lax.axis_index("s")
    wid = sid * NC + cid
    idx = [idx0, idx1]
    pltpu.sync_copy(zeros_hbm, rows_v)
    _zero_rows(rows_v, hist_sh, sid * rps, rps)
    pltpu.sync_copy(ones_hbm, rows_v)
    plsc.subcore_barrier()
    pltpu.sync_copy(dst_hbm.at[wid, 0], idx0)

    def scatter_group(idx_v):
      descs = [
          pltpu.async_copy(rows_v, hist_sh.at[idx_v.at[k]], sem, add=True)
          for k in range(G)
      ]
      for dsc in descs:
        dsc.wait()

    def body(hi, carry):
      gi = 2 * hi
      pf1 = pltpu.async_copy(dst_hbm.at[wid, gi + 1], idx1, isem)
      scatter_group(idx0)
      pf1.wait()
      nxt = lax.min(gi + 2, ng - 1)
      pf0 = pltpu.async_copy(dst_hbm.at[wid, nxt], idx0, isem)
      scatter_group(idx1)
      pf0.wait()
      return carry

    lax.fori_loop(0, ng // 2, body, 0)
    plsc.subcore_barrier()
    pltpu.sync_copy(hist_sh.at[pl.ds(sid * rps, rps)], out_hbm.at[cid, sid])

  return hist_kernel


def _make_agg_kernel(n_nodes, ng, nch, d):
  rps = n_nodes // NS

  @functools.partial(
      pl.kernel,
      out_type=[jax.ShapeDtypeStruct((NC, NS, rps, d), jnp.float32)] * nch,
      mesh=_sc_mesh(),
      scratch_types=[
          pltpu.VMEM((G, B), jnp.int32),
          pltpu.VMEM((G, B), jnp.int32),
          pltpu.VMEM((G, B), jnp.int32),
          pltpu.VMEM((G, B), jnp.int32),
          pltpu.VMEM((B, d), jnp.float32),
          pltpu.VMEM((B, d), jnp.float32),
          pltpu.VMEM_SHARED((n_nodes, d), jnp.float32),
          pltpu.SemaphoreType.DMA,
          pltpu.SemaphoreType.DMA,
          pltpu.SemaphoreType.DMA,
          pltpu.SemaphoreType.DMA,
      ],
  )
  def agg_kernel(*refs):
    g_refs = refs[:nch]
    src_hbm, dst_hbm, zeros_hbm = refs[nch:nch + 3]
    out_refs = refs[nch + 3:2 * nch + 3]
    (idxs0, idxs1, idxd0, idxd1, rows0, rows1, acc_sh, gsem, ssem0, ssem1,
     isem) = refs[2 * nch + 3:]
    rows = [rows0, rows1]
    ssems = [ssem0, ssem1]
    idxs = [idxs0, idxs1]
    idxd = [idxd0, idxd1]
    cid = lax.axis_index("c")
    sid = lax.axis_index("s")
    wid = sid * NC + cid

    def process_group(g_ref, idxs_v, idxd_v):
      gd = [None] * G
      sd = [None, None]
      gd[0] = pltpu.async_copy(g_ref.at[idxs_v.at[0]], rows[0], gsem)
      for k in range(G):
        p = k % 2
        gd[k].wait()
        sd[p] = pltpu.async_copy(rows[p], acc_sh.at[idxd_v.at[k]],
                                 ssems[p], add=True)
        if k + 1 < G:
          q = (k + 1) % 2
          if sd[q] is not None:
            sd[q].wait()
          gd[k + 1] = pltpu.async_copy(g_ref.at[idxs_v.at[k + 1]],
                                       rows[q], gsem)
      sd[0].wait()
      sd[1].wait()

    for c in range(nch):
      pltpu.sync_copy(zeros_hbm, rows0)
      _zero_rows(rows0, acc_sh, sid * rps, rps)
      plsc.subcore_barrier()
      pltpu.sync_copy(src_hbm.at[wid, 0], idxs0)
      pltpu.sync_copy(dst_hbm.at[wid, 0], idxd0)

      def body(hi, carry):
        gi = 2 * hi
        pf = [pltpu.async_copy(src_hbm.at[wid, gi + 1], idxs1, isem),
              pltpu.async_copy(dst_hbm.at[wid, gi + 1], idxd1, isem)]
        process_group(g_refs[c], idxs0, idxd0)
        for d_ in pf:
          d_.wait()
        nxt = lax.min(gi + 2, ng - 1)
        pf = [pltpu.async_copy(src_hbm.at[wid, nxt], idxs0, isem),
              pltpu.async_copy(dst_hbm.at[wid, nxt], idxd0, isem)]
        process_group(g_refs[c], idxs1, idxd1)
        for d_ in pf:
          d_.wait()
        return carry

      lax.fori_loop(0, ng // 2, body, 0)
      plsc.subcore_barrier()
      pltpu.sync_copy(acc_sh.at[pl.ds(sid * rps, rps)],
                      out_refs[c].at[cid, sid])
      plsc.subcore_barrier()

  return agg_kernel


def _g1_body(hist_ref, x_ref, w1_ref, ga_ref, gb_ref, dinv_ref):
  deg = hist_ref[0] + hist_ref[1]
  dinv = lax.rsqrt(deg[:, 0:1] + 1.0)
  dinv_ref[...] = dinv
  h = jnp.dot(x_ref[...], w1_ref[...], preferred_element_type=jnp.float32)
  ga_ref[...] = h[:, :128] * dinv
  gb_ref[...] = h[:, 128:] * dinv


def _g2_body(dinv_ref, a0_ref, a1_ref, ga_ref, gb_ref, w2_ref, b1_ref,
             g2_ref):
  dinv = dinv_ref[...]
  h0 = jnp.maximum(dinv * (a0_ref[0] + a0_ref[1] + ga_ref[...])
                   + b1_ref[0:1, :128], 0.0)
  h1 = jnp.maximum(dinv * (a1_ref[0] + a1_ref[1] + gb_ref[...])
                   + b1_ref[0:1, 128:], 0.0)
  g2_ref[...] = dinv * (
      jnp.dot(h0, w2_ref[0], preferred_element_type=jnp.float32)
      + jnp.dot(h1, w2_ref[1], preferred_element_type=jnp.float32))


def _out_body(dinv_ref, a2_ref, g2_ref, b2_ref, out_ref):
  out_ref[...] = dinv_ref[...] * (a2_ref[0] + a2_ref[1] + g2_ref[...]) \
      + b2_ref[...]


def kernel(x, edge_index, W1, b1, W2, b2):
  n, d_in = x.shape
  d_hid = W1.shape[1]
  d_out = W2.shape[1]
  e = edge_index.shape[1]
  assert e % (NW * B * G) == 0 and n % NS == 0
  assert d_in == 128 and d_hid == 256 and d_out == 128
  ng = e // (NW * B * G)
  assert ng % 2 == 0
  rb = 1000
  nblk = n // rb

  src = edge_index[0].reshape(NW, ng, G, B)
  dst = edge_index[1].reshape(NW, ng, G, B)
  ones_h = jnp.ones((B, HW), jnp.float32)
  zeros_d = jnp.zeros((B, 128), jnp.float32)
  w2r = W2.reshape(2, 128, d_out)
  b1r = b1.reshape(1, d_hid)
  b2r = b2.reshape(1, d_out)

  hist = _make_hist_kernel(n, ng)(dst, ones_h, zeros_d).reshape(NC, n, HW)

  hist_spec = pl.BlockSpec((NC, rb, HW), lambda i: (0, i, 0))
  row_spec = pl.BlockSpec((rb, 128), lambda i: (i, 0))
  agg_spec = pl.BlockSpec((NC, rb, 128), lambda i: (0, i, 0))
  dinv_spec = pl.BlockSpec((rb, 1), lambda i: (i, 0))

  g1a, g1b, dinv = pl.pallas_call(
      _g1_body,
      grid=(nblk,),
      in_specs=[
          hist_spec,
          pl.BlockSpec((rb, d_in), lambda i: (i, 0)),
          pl.BlockSpec((d_in, d_hid), lambda i: (0, 0)),
      ],
      out_specs=[row_spec, row_spec, dinv_spec],
      out_shape=[jax.ShapeDtypeStruct((n, 128), jnp.float32)] * 2
      + [jax.ShapeDtypeStruct((n, 1), jnp.float32)],
  )(hist, x, W1)

  a1a, a1b = _make_agg_kernel(n, ng, 2, 128)(g1a, g1b, src, dst, zeros_d)
  a1a = a1a.reshape(NC, n, 128)
  a1b = a1b.reshape(NC, n, 128)

  g2 = pl.pallas_call(
      _g2_body,
      grid=(nblk,),
      in_specs=[
          dinv_spec, agg_spec, agg_spec, row_spec, row_spec,
          pl.BlockSpec((2, 128, d_out), lambda i: (0, 0, 0)),
          pl.BlockSpec((1, d_hid), lambda i: (0, 0)),
      ],
      out_specs=row_spec,
      out_shape=jax.ShapeDtypeStruct((n, 128), jnp.float32),
  )(dinv, a1a, a1b, g1a, g1b, w2r, b1r)

  (a2,) = _make_agg_kernel(n, ng, 1, 128)(g2, src, dst, zeros_d)
  a2 = a2.reshape(NC, n, 128)

  out = pl.pallas_call(
      _out_body,
      grid=(nblk,),
      in_specs=[
          dinv_spec, agg_spec, row_spec,
          pl.BlockSpec((1, d_out), lambda i: (0, 0)),
      ],
      out_specs=pl.BlockSpec((rb, d_out), lambda i: (i, 0)),
      out_shape=jax.ShapeDtypeStruct((n, d_out), jnp.float32),
  )(dinv, a2, g2, b2r)
  return out


# speedup vs baseline: 21.7982x; 1.2012x over previous
import functools

import jax
import jax.numpy as jnp
from jax import lax
from jax.experimental import pallas as pl
from jax.experimental.pallas import tpu as pltpu
from jax.experimental.pallas import tpu_sc as plsc

NC = 2
NS = 16
NW = NC * NS
B = 100
G = 10
HW = 128

@functools.cache
def _sc_mesh():
  return plsc.VectorSubcoreMesh(
      core_axis_name="c", subcore_axis_name="s", num_cores=NC, num_subcores=NS
  )


def _zero_rows(zeros_v, acc_sh, base, rps):
  full, rem = divmod(rps, B)
  for k in range(full):
    pltpu.sync_copy(zeros_v, acc_sh.at[pl.ds(base + k * B, B)])
  if rem:
    pltpu.sync_copy(zeros_v.at[pl.ds(0, rem)],
                    acc_sh.at[pl.ds(base + full * B, rem)])


def _make_hist_kernel(n_nodes, ng):
  rps = n_nodes // NS

  @functools.partial(
      pl.kernel,
      out_type=jax.ShapeDtypeStruct((NC, NS, rps, HW), jnp.float32),
      mesh=_sc_mesh(),
      scratch_types=[
          pltpu.VMEM((G, B), jnp.int32),
          pltpu.VMEM((G, B), jnp.int32),
          pltpu.VMEM((B, HW), jnp.float32),
          pltpu.VMEM_SHARED((n_nodes, HW), jnp.float32),
          pltpu.SemaphoreType.DMA,
          pltpu.SemaphoreType.DMA,
      ],
  )
  def hist_kernel(dst_hbm, ones_hbm, zeros_hbm, out_hbm, idx0, idx1, rows_v,
                  hist_sh, sem, isem):
    cid = lax.axis_index("c")
    sid = lax.axis_index("s")
    wid = sid * NC + cid
    idx = [idx0, idx1]
    pltpu.sync_copy(zeros_hbm, rows_v)
    _zero_rows(rows_v, hist_sh, sid * rps, rps)
    pltpu.sync_copy(ones_hbm, rows_v)
    plsc.subcore_barrier()
    pltpu.sync_copy(dst_hbm.at[wid, 0], idx0)

    def scatter_group(idx_v):
      descs = [
          pltpu.async_copy(rows_v, hist_sh.at[idx_v.at[k]], sem, add=True)
          for k in range(G)
      ]
      for dsc in descs:
        dsc.wait()

    def body(hi, carry):
      gi = 2 * hi
      pf1 = pltpu.async_copy(dst_hbm.at[wid, gi + 1], idx1, isem)
      scatter_group(idx0)
      pf1.wait()
      nxt = lax.min(gi + 2, ng - 1)
      pf0 = pltpu.async_copy(dst_hbm.at[wid, nxt], idx0, isem)
      scatter_group(idx1)
      pf0.wait()
      return carry

    lax.fori_loop(0, ng // 2, body, 0)
    plsc.subcore_barrier()
    pltpu.sync_copy(hist_sh.at[pl.ds(sid * rps, rps)], out_hbm.at[cid, sid])

  return hist_kernel


def _make_agg_kernel(n_nodes, ng, nch, d):
  rps = n_nodes // NS

  @functools.partial(
      pl.kernel,
      out_type=[jax.ShapeDtypeStruct((NC, NS, rps, d), jnp.float32)] * nch,
      mesh=_sc_mesh(),
      scratch_types=[
          pltpu.VMEM((G, B), jnp.int32),
          pltpu.VMEM((G, B), jnp.int32),
          pltpu.VMEM((G, B), jnp.int32),
          pltpu.VMEM((G, B), jnp.int32),
          pltpu.VMEM((B, d), jnp.float32),
          pltpu.VMEM((B, d), jnp.float32),
          pltpu.VMEM((B, d), jnp.float32),
          pltpu.VMEM_SHARED((n_nodes, d), jnp.float32),
          [pltpu.SemaphoreType.DMA] * 3,
          [pltpu.SemaphoreType.DMA] * 3,
          pltpu.SemaphoreType.DMA,
      ],
  )
  def agg_kernel(*refs):
    g_refs = refs[:nch]
    src_hbm, dst_hbm, zeros_hbm = refs[nch:nch + 3]
    out_refs = refs[nch + 3:2 * nch + 3]
    (idxs0, idxs1, idxd0, idxd1, r0, r1, r2, acc_sh, gsems, ssems,
     isem) = refs[2 * nch + 3:]
    rows = [r0, r1, r2]
    isv = [idxs0, idxs1]
    idv = [idxd0, idxd1]
    cid = lax.axis_index("c")
    sid = lax.axis_index("s")
    wid = sid * NC + cid
    T = 2 * G

    def process_pair(g_ref, gi):
      def idx_of(j):
        return isv[j // G].at[j % G], idv[j // G].at[j % G]

      pf = [pltpu.async_copy(src_hbm.at[wid, gi + 1], idxs1, isem),
            pltpu.async_copy(dst_hbm.at[wid, gi + 1], idxd1, isem)]
      gd = [None] * T
      sd = [None] * T
      for j in range(2):
        s_idx, _ = idx_of(j)
        gd[j] = pltpu.async_copy(g_ref.at[s_idx], rows[j % 3],
                                 gsems[j % 3])
      for j in range(T):
        gd[j].wait()
        _, d_idx = idx_of(j)
        sd[j] = pltpu.async_copy(rows[j % 3], acc_sh.at[d_idx],
                                 ssems[j % 3], add=True)
        if j + 2 < T:
          if j + 2 == G:
            for d_ in pf:
              d_.wait()
          if j >= 1:
            sd[j - 1].wait()
          s_idx, _ = idx_of(j + 2)
          gd[j + 2] = pltpu.async_copy(g_ref.at[s_idx], rows[(j + 2) % 3],
                                       gsems[(j + 2) % 3])
      for j in range(T - 3, T):
        sd[j].wait()

    for c in range(nch):
      pltpu.sync_copy(zeros_hbm, r0)
      _zero_rows(r0, acc_sh, sid * rps, rps)
      plsc.subcore_barrier()

      def body(hi, carry):
        gi = 2 * hi
        pltpu.sync_copy(src_hbm.at[wid, gi], idxs0)
        pltpu.sync_copy(dst_hbm.at[wid, gi], idxd0)
        process_pair(g_refs[c], gi)
        return carry

      lax.fori_loop(0, ng // 2, body, 0)
      plsc.subcore_barrier()
      pltpu.sync_copy(acc_sh.at[pl.ds(sid * rps, rps)],
                      out_refs[c].at[cid, sid])
      plsc.subcore_barrier()

  return agg_kernel


def _g1_body(hist_ref, x_ref, w1_ref, ga_ref, gb_ref, dinv_ref):
  deg = hist_ref[0] + hist_ref[1]
  dinv = lax.rsqrt(deg[:, 0:1] + 1.0)
  dinv_ref[...] = dinv
  h = jnp.dot(x_ref[...], w1_ref[...], preferred_element_type=jnp.float32)
  ga_ref[...] = h[:, :128] * dinv
  gb_ref[...] = h[:, 128:] * dinv


def _g2_body(dinv_ref, a0_ref, a1_ref, ga_ref, gb_ref, w2_ref, b1_ref,
             g2_ref):
  dinv = dinv_ref[...]
  h0 = jnp.maximum(dinv * (a0_ref[0] + a0_ref[1] + ga_ref[...])
                   + b1_ref[0:1, :128], 0.0)
  h1 = jnp.maximum(dinv * (a1_ref[0] + a1_ref[1] + gb_ref[...])
                   + b1_ref[0:1, 128:], 0.0)
  g2_ref[...] = dinv * (
      jnp.dot(h0, w2_ref[0], preferred_element_type=jnp.float32)
      + jnp.dot(h1, w2_ref[1], preferred_element_type=jnp.float32))


def _out_body(dinv_ref, a2_ref, g2_ref, b2_ref, out_ref):
  out_ref[...] = dinv_ref[...] * (a2_ref[0] + a2_ref[1] + g2_ref[...]) \
      + b2_ref[...]


def kernel(x, edge_index, W1, b1, W2, b2):
  n, d_in = x.shape
  d_hid = W1.shape[1]
  d_out = W2.shape[1]
  e = edge_index.shape[1]
  assert e % (NW * B * G) == 0 and n % NS == 0
  assert d_in == 128 and d_hid == 256 and d_out == 128
  ng = e // (NW * B * G)
  assert ng % 2 == 0
  rb = 1000
  nblk = n // rb

  src = edge_index[0].reshape(NW, ng, G, B)
  dst = edge_index[1].reshape(NW, ng, G, B)
  ones_h = jnp.ones((B, HW), jnp.float32)
  zeros_d = jnp.zeros((B, 128), jnp.float32)
  w2r = W2.reshape(2, 128, d_out)
  b1r = b1.reshape(1, d_hid)
  b2r = b2.reshape(1, d_out)

  hist = _make_hist_kernel(n, ng)(dst, ones_h, zeros_d).reshape(NC, n, HW)

  hist_spec = pl.BlockSpec((NC, rb, HW), lambda i: (0, i, 0))
  row_spec = pl.BlockSpec((rb, 128), lambda i: (i, 0))
  agg_spec = pl.BlockSpec((NC, rb, 128), lambda i: (0, i, 0))
  dinv_spec = pl.BlockSpec((rb, 1), lambda i: (i, 0))

  g1a, g1b, dinv = pl.pallas_call(
      _g1_body,
      grid=(nblk,),
      in_specs=[
          hist_spec,
          pl.BlockSpec((rb, d_in), lambda i: (i, 0)),
          pl.BlockSpec((d_in, d_hid), lambda i: (0, 0)),
      ],
      out_specs=[row_spec, row_spec, dinv_spec],
      out_shape=[jax.ShapeDtypeStruct((n, 128), jnp.float32)] * 2
      + [jax.ShapeDtypeStruct((n, 1), jnp.float32)],
  )(hist, x, W1)

  a1a, a1b = _make_agg_kernel(n, ng, 2, 128)(g1a, g1b, src, dst, zeros_d)
  a1a = a1a.reshape(NC, n, 128)
  a1b = a1b.reshape(NC, n, 128)

  g2 = pl.pallas_call(
      _g2_body,
      grid=(nblk,),
      in_specs=[
          dinv_spec, agg_spec, agg_spec, row_spec, row_spec,
          pl.BlockSpec((2, 128, d_out), lambda i: (0, 0, 0)),
          pl.BlockSpec((1, d_hid), lambda i: (0, 0)),
      ],
      out_specs=row_spec,
      out_shape=jax.ShapeDtypeStruct((n, 128), jnp.float32),
  )(dinv, a1a, a1b, g1a, g1b, w2r, b1r)

  (a2,) = _make_agg_kernel(n, ng, 1, 128)(g2, src, dst, zeros_d)
  a2 = a2.reshape(NC, n, 128)

  out = pl.pallas_call(
      _out_body,
      grid=(nblk,),
      in_specs=[
          dinv_spec, agg_spec, row_spec,
          pl.BlockSpec((1, d_out), lambda i: (0, 0)),
      ],
      out_specs=pl.BlockSpec((rb, d_out), lambda i: (i, 0)),
      out_shape=jax.ShapeDtypeStruct((n, d_out), jnp.float32),
  )(dinv, a2, g2, b2r)
  return out


# speedup vs baseline: 21.8764x; 1.0036x over previous
import functools

import jax
import jax.numpy as jnp
from jax import lax
from jax.experimental import pallas as pl
from jax.experimental.pallas import tpu as pltpu
from jax.experimental.pallas import tpu_sc as plsc

NC = 2
NS = 16
NW = NC * NS
B = 100
G = 10
HW = 128

@functools.cache
def _sc_mesh():
  return plsc.VectorSubcoreMesh(
      core_axis_name="c", subcore_axis_name="s", num_cores=NC, num_subcores=NS
  )


def _zero_rows(zeros_v, acc_sh, base, rps, sem=None):
  full, rem = divmod(rps, B)
  descs = []
  for k in range(full):
    dst = acc_sh.at[pl.ds(base + k * B, B)]
    if sem is None:
      pltpu.sync_copy(zeros_v, dst)
    else:
      descs.append(pltpu.async_copy(zeros_v, dst, sem))
  if rem:
    dst = acc_sh.at[pl.ds(base + full * B, rem)]
    src = zeros_v.at[pl.ds(0, rem)]
    if sem is None:
      pltpu.sync_copy(src, dst)
    else:
      descs.append(pltpu.async_copy(src, dst, sem))
  for dsc in descs:
    dsc.wait()


def _make_hist_kernel(n_nodes, ng):
  rps = n_nodes // NS

  @functools.partial(
      pl.kernel,
      out_type=jax.ShapeDtypeStruct((NC, NS, rps, HW), jnp.float32),
      mesh=_sc_mesh(),
      scratch_types=[
          pltpu.VMEM((G, B), jnp.int32),
          pltpu.VMEM((G, B), jnp.int32),
          pltpu.VMEM((B, HW), jnp.float32),
          pltpu.VMEM_SHARED((n_nodes, HW), jnp.float32),
          pltpu.SemaphoreType.DMA,
          pltpu.SemaphoreType.DMA,
      ],
  )
  def hist_kernel(dst_hbm, ones_hbm, zeros_hbm, out_hbm, idx0, idx1, rows_v,
                  hist_sh, sem, isem):
    cid = lax.axis_index("c")
    sid = lax.axis_index("s")
    wid = sid * NC + cid
    pltpu.sync_copy(zeros_hbm, rows_v)
    _zero_rows(rows_v, hist_sh, sid * rps, rps)
    pltpu.sync_copy(ones_hbm, rows_v)
    plsc.subcore_barrier()
    pltpu.sync_copy(dst_hbm.at[wid, 0], idx0)

    def scatter_group(idx_v):
      descs = [
          pltpu.async_copy(rows_v, hist_sh.at[idx_v.at[k]], sem, add=True)
          for k in range(G)
      ]
      for dsc in descs:
        dsc.wait()

    def body(hi, carry):
      gi = 2 * hi
      pf1 = pltpu.async_copy(dst_hbm.at[wid, gi + 1], idx1, isem)
      scatter_group(idx0)
      pf1.wait()
      nxt = lax.min(gi + 2, ng - 1)
      pf0 = pltpu.async_copy(dst_hbm.at[wid, nxt], idx0, isem)
      scatter_group(idx1)
      pf0.wait()
      return carry

    lax.fori_loop(0, ng // 2, body, 0)
    plsc.subcore_barrier()
    pltpu.sync_copy(hist_sh.at[pl.ds(sid * rps, rps)], out_hbm.at[cid, sid])

  return hist_kernel


def _make_agg_kernel(n_nodes, ng, nch, d):
  rps = n_nodes // NS

  @functools.partial(
      pl.kernel,
      out_type=[jax.ShapeDtypeStruct((NC, NS, rps, d), jnp.float32)] * nch,
      mesh=_sc_mesh(),
      scratch_types=[
          pltpu.VMEM((G, B), jnp.int32),
          pltpu.VMEM((G, B), jnp.int32),
          pltpu.VMEM((G, B), jnp.int32),
          pltpu.VMEM((G, B), jnp.int32),
          pltpu.VMEM((B, d), jnp.float32),
          pltpu.VMEM((B, d), jnp.float32),
          pltpu.VMEM((B, d), jnp.float32),
          pltpu.VMEM_SHARED((n_nodes, d), jnp.float32),
          [pltpu.SemaphoreType.DMA] * 3,
          [pltpu.SemaphoreType.DMA] * 3,
          pltpu.SemaphoreType.DMA,
      ],
  )
  def agg_kernel(*refs):
    g_refs = refs[:nch]
    src_hbm, dst_hbm, zeros_hbm = refs[nch:nch + 3]
    out_refs = refs[nch + 3:2 * nch + 3]
    (idxs0, idxs1, idxd0, idxd1, r0, r1, r2, acc_sh, gsems, ssems,
     isem) = refs[2 * nch + 3:]
    rows = [r0, r1, r2]
    isv = [idxs0, idxs1]
    idv = [idxd0, idxd1]
    cid = lax.axis_index("c")
    sid = lax.axis_index("s")
    wid = sid * NC + cid
    T = 2 * G

    def process_pair(g_ref, gi):
      def idx_of(j):
        return isv[j // G].at[j % G], idv[j // G].at[j % G]

      pf = [pltpu.async_copy(src_hbm.at[wid, gi + 1], idxs1, isem),
            pltpu.async_copy(dst_hbm.at[wid, gi + 1], idxd1, isem)]
      gd = [None] * T
      sd = [None] * T
      for j in range(2):
        s_idx, _ = idx_of(j)
        gd[j] = pltpu.async_copy(g_ref.at[s_idx], rows[j % 3],
                                 gsems[j % 3])
      for j in range(T):
        gd[j].wait()
        _, d_idx = idx_of(j)
        sd[j] = pltpu.async_copy(rows[j % 3], acc_sh.at[d_idx],
                                 ssems[j % 3], add=True)
        if j + 2 < T:
          if j + 2 == G:
            for d_ in pf:
              d_.wait()
          if j >= 1:
            sd[j - 1].wait()
          s_idx, _ = idx_of(j + 2)
          gd[j + 2] = pltpu.async_copy(g_ref.at[s_idx], rows[(j + 2) % 3],
                                       gsems[(j + 2) % 3])
      for j in range(T - 3, T):
        sd[j].wait()

    for c in range(nch):
      pltpu.sync_copy(zeros_hbm, r0)
      _zero_rows(r0, acc_sh, sid * rps, rps, sem=isem)
      plsc.subcore_barrier()

      def body(hi, carry):
        gi = 2 * hi
        pltpu.sync_copy(src_hbm.at[wid, gi], idxs0)
        pltpu.sync_copy(dst_hbm.at[wid, gi], idxd0)
        process_pair(g_refs[c], gi)
        return carry

      lax.fori_loop(0, ng // 2, body, 0)
      plsc.subcore_barrier()
      pltpu.sync_copy(acc_sh.at[pl.ds(sid * rps, rps)],
                      out_refs[c].at[cid, sid])

  return agg_kernel


def _g1_body(hist_ref, x_ref, w1_ref, ga_ref, gb_ref, dinv_ref):
  deg = hist_ref[0] + hist_ref[1]
  dinv = lax.rsqrt(deg[:, 0:1] + 1.0)
  dinv_ref[...] = dinv
  h = jnp.dot(x_ref[...], w1_ref[...], preferred_element_type=jnp.float32)
  ga_ref[...] = h[:, :128] * dinv
  gb_ref[...] = h[:, 128:] * dinv


def _g2_body(dinv_ref, a0_ref, a1_ref, ga_ref, gb_ref, w2_ref, b1_ref,
             g2_ref):
  dinv = dinv_ref[...]
  h0 = jnp.maximum(dinv * (a0_ref[0] + a0_ref[1] + ga_ref[...])
                   + b1_ref[0:1, :128], 0.0)
  h1 = jnp.maximum(dinv * (a1_ref[0] + a1_ref[1] + gb_ref[...])
                   + b1_ref[0:1, 128:], 0.0)
  g2_ref[...] = dinv * (
      jnp.dot(h0, w2_ref[0], preferred_element_type=jnp.float32)
      + jnp.dot(h1, w2_ref[1], preferred_element_type=jnp.float32))


def _out_body(dinv_ref, a2_ref, g2_ref, b2_ref, out_ref):
  out_ref[...] = dinv_ref[...] * (a2_ref[0] + a2_ref[1] + g2_ref[...]) \
      + b2_ref[...]


def kernel(x, edge_index, W1, b1, W2, b2):
  n, d_in = x.shape
  d_hid = W1.shape[1]
  d_out = W2.shape[1]
  e = edge_index.shape[1]
  assert e % (NW * B * G) == 0 and n % NS == 0
  assert d_in == 128 and d_hid == 256 and d_out == 128
  ng = e // (NW * B * G)
  assert ng % 2 == 0
  rb = 1000
  nblk = n // rb

  src = edge_index[0].reshape(NW, ng, G, B)
  dst = edge_index[1].reshape(NW, ng, G, B)
  zeros_d = jnp.zeros((B, 128), jnp.float32)
  w2r = W2.reshape(2, 128, d_out)
  b1r = b1.reshape(1, d_hid)
  b2r = b2.reshape(1, d_out)
  rps = n // NS

  ones_h = jnp.ones((B, HW), jnp.float32)
  hist = _make_hist_kernel(n, ng)(dst, ones_h, zeros_d).reshape(NC, n, HW)

  hist_spec = pl.BlockSpec((NC, rb, HW), lambda i: (0, i, 0))
  row_spec = pl.BlockSpec((rb, 128), lambda i: (i, 0))
  agg_spec = pl.BlockSpec((NC, rb, 128), lambda i: (0, i, 0))
  dinv_spec = pl.BlockSpec((rb, 1), lambda i: (i, 0))

  g1a, g1b, dinv = pl.pallas_call(
      _g1_body,
      grid=(nblk,),
      in_specs=[
          hist_spec,
          pl.BlockSpec((rb, d_in), lambda i: (i, 0)),
          pl.BlockSpec((d_in, d_hid), lambda i: (0, 0)),
      ],
      out_specs=[row_spec, row_spec, dinv_spec],
      out_shape=[jax.ShapeDtypeStruct((n, 128), jnp.float32)] * 2
      + [jax.ShapeDtypeStruct((n, 1), jnp.float32)],
  )(hist, x, W1)

  a1a, a1b = _make_agg_kernel(n, ng, 2, 128)(g1a, g1b, src, dst, zeros_d)
  a1a = a1a.reshape(NC, n, 128)
  a1b = a1b.reshape(NC, n, 128)

  g2 = pl.pallas_call(
      _g2_body,
      grid=(nblk,),
      in_specs=[
          dinv_spec, agg_spec, agg_spec, row_spec, row_spec,
          pl.BlockSpec((2, 128, d_out), lambda i: (0, 0, 0)),
          pl.BlockSpec((1, d_hid), lambda i: (0, 0)),
      ],
      out_specs=row_spec,
      out_shape=jax.ShapeDtypeStruct((n, 128), jnp.float32),
  )(dinv, a1a, a1b, g1a, g1b, w2r, b1r)

  (a2,) = _make_agg_kernel(n, ng, 1, 128)(g2, src, dst, zeros_d)
  a2 = a2.reshape(NC, n, 128)

  out = pl.pallas_call(
      _out_body,
      grid=(nblk,),
      in_specs=[
          dinv_spec, agg_spec, row_spec,
          pl.BlockSpec((1, d_out), lambda i: (0, 0)),
      ],
      out_specs=pl.BlockSpec((rb, d_out), lambda i: (i, 0)),
      out_shape=jax.ShapeDtypeStruct((n, d_out), jnp.float32),
  )(dinv, a2, g2, b2r)
  return out
